# baseline (speedup 1.0000x reference)
import jax
import jax.numpy as jnp
from jax import lax
from jax.experimental import pallas as pl
from jax.experimental.pallas import tpu as pltpu
from jax.experimental.pallas import tpu_sc as plsc

B = 4
S = 4096
D = 1024
NW = 32
ROWS_W = S // NW
CHUNK = 4
NCHUNK = ROWS_W // CHUNK
NBUF = 4
LANES = 16
GROUP = 8


def _compute(xb, pb):
    n_cc = D // (GROUP * LANES)
    cc_shift = n_cc.bit_length() - 1

    def body(j, _):
        pr = j >> cc_shift
        c0 = (j & (n_cc - 1)) * (GROUP * LANES)
        sls = [pl.ds(pl.multiple_of(c0 + k * LANES, LANES), LANES)
               for k in range(GROUP)]
        ps = [pb[pr, sl] for sl in sls]
        for b in range(B):
            xs = [xb[b, pr, sl] for sl in sls]
            for k in range(GROUP):
                xb[b, pr, sls[k]] = xs[k] + ps[k]
        return 0

    lax.fori_loop(0, CHUNK * n_cc, body, 0)


def _sc_add(x_hbm, p_hbm, o_hbm, *scr):
    xbufs = scr[:NBUF]
    pbufs = scr[NBUF:2 * NBUF]
    sin = scr[2 * NBUF:3 * NBUF]
    sout = scr[3 * NBUF:]
    wid = lax.axis_index("s") * 2 + lax.axis_index("c")
    s0 = wid * ROWS_W

    def start_in(c):
        buf = c % NBUF
        srow = s0 + c * CHUNK
        return [
            pltpu.async_copy(p_hbm.at[pl.ds(srow, CHUNK)],
                             pbufs[buf], sin[buf]),
            pltpu.async_copy(x_hbm.at[:, pl.ds(srow, CHUNK), :],
                             xbufs[buf], sin[buf]),
        ]

    def start_out(c):
        buf = c % NBUF
        srow = s0 + c * CHUNK
        return [
            pltpu.async_copy(xbufs[buf],
                             o_hbm.at[:, pl.ds(srow, CHUNK), :], sout[buf]),
        ]

    pend_in = {0: start_in(0), 1: start_in(1), 2: start_in(2)}
    pend_out = {}
    for c in range(NCHUNK):
        buf = c % NBUF
        for cp in pend_in.pop(c):
            cp.wait()
        _compute(xbufs[buf], pbufs[buf])
        pend_out[c] = start_out(c)
        if c + 3 < NCHUNK:
            if c >= 1:
                for cp in pend_out.pop(c - 1):
                    cp.wait()
            pend_in[c + 3] = start_in(c + 3)
    for c, cps in sorted(pend_out.items()):
        for cp in cps:
            cp.wait()


@jax.jit
def _sc_kernel(x, p2d):
    mesh = plsc.VectorSubcoreMesh(core_axis_name="c", subcore_axis_name="s")
    run = pl.kernel(
        _sc_add,
        out_type=jax.ShapeDtypeStruct((B, S, D), jnp.float32),
        mesh=mesh,
        scratch_types=(
            [pltpu.VMEM((B, CHUNK, D), jnp.float32)] * NBUF
            + [pltpu.VMEM((CHUNK, D), jnp.float32)] * NBUF
            + [pltpu.SemaphoreType.DMA] * (2 * NBUF)
        ),
    )
    return run(x, p2d)


def kernel(x, pos_table):
    return _sc_kernel(x, pos_table)


# speedup vs baseline: 1.0536x; 1.0536x over previous
import jax
import jax.numpy as jnp
from jax import lax
from jax.experimental import pallas as pl
from jax.experimental.pallas import tpu as pltpu
from jax.experimental.pallas import tpu_sc as plsc

B = 4
S = 4096
D = 1024
NW = 32
ROWS_W = S // NW
CHUNK = 8
NCHUNK = ROWS_W // CHUNK
NBUF = 3
LANES = 16
GROUP = 16


def _compute(xb, pb):
    n_cc = D // (GROUP * LANES)

    def body(j, _):
        pr = j >> 2
        c0 = (j & (n_cc - 1)) * (GROUP * LANES)
        sls = [pl.ds(pl.multiple_of(c0 + k * LANES, LANES), LANES)
               for k in range(GROUP)]
        ps = [pb[pr, sl] for sl in sls]
        for b in range(B):
            xs = [xb[b, pr, sl] for sl in sls]
            for k in range(GROUP):
                xb[b, pr, sls[k]] = xs[k] + ps[k]
        return 0

    lax.fori_loop(0, CHUNK * n_cc, body, 0)


def _sc_add(x_hbm, p_hbm, o_hbm, *scr):
    xbufs = scr[:NBUF]
    pbufs = scr[NBUF:2 * NBUF]
    sin = scr[2 * NBUF:3 * NBUF]
    sout = scr[3 * NBUF:]
    wid = lax.axis_index("s") * 2 + lax.axis_index("c")
    s0 = wid * ROWS_W

    def start_in(c):
        buf = c % NBUF
        srow = s0 + c * CHUNK
        return [
            pltpu.async_copy(p_hbm.at[pl.ds(srow, CHUNK)],
                             pbufs[buf], sin[buf]),
            pltpu.async_copy(x_hbm.at[:, pl.ds(srow, CHUNK), :],
                             xbufs[buf], sin[buf]),
        ]

    def start_out(c):
        buf = c % NBUF
        srow = s0 + c * CHUNK
        return [
            pltpu.async_copy(xbufs[buf],
                             o_hbm.at[:, pl.ds(srow, CHUNK), :], sout[buf]),
        ]

    pend_in = {0: start_in(0), 1: start_in(1)}
    pend_out = {}
    for c in range(NCHUNK):
        buf = c % NBUF
        for cp in pend_in.pop(c):
            cp.wait()
        _compute(xbufs[buf], pbufs[buf])
        pend_out[c] = start_out(c)
        if c + 2 < NCHUNK:
            if c >= 1:
                for cp in pend_out.pop(c - 1):
                    cp.wait()
            pend_in[c + 2] = start_in(c + 2)
    for c, cps in sorted(pend_out.items()):
        for cp in cps:
            cp.wait()


@jax.jit
def _sc_kernel(x, p2d):
    mesh = plsc.VectorSubcoreMesh(core_axis_name="c", subcore_axis_name="s")
    run = pl.kernel(
        _sc_add,
        out_type=jax.ShapeDtypeStruct((B, S, D), jnp.float32),
        mesh=mesh,
        scratch_types=(
            [pltpu.VMEM((B, CHUNK, D), jnp.float32)] * NBUF
            + [pltpu.VMEM((CHUNK, D), jnp.float32)] * NBUF
            + [pltpu.SemaphoreType.DMA] * (2 * NBUF)
        ),
    )
    return run(x, p2d)


def kernel(x, pos_table):
    return _sc_kernel(x, pos_table)


# speedup vs baseline: 1.0816x; 1.0266x over previous
import jax
import jax.numpy as jnp
from jax import lax
from jax.experimental import pallas as pl
from jax.experimental.pallas import tpu as pltpu
from jax.experimental.pallas import tpu_sc as plsc

B = 4
S = 4096
D = 1024
NW = 32
ROWS_W = S // NW
CHUNK = 8
NCHUNK = ROWS_W // CHUNK
NBUF = 3
LANES = 16
GROUP = 16


def _compute(xb, pb):
    n_cc = D // (GROUP * LANES)

    def body(j, _):
        pr = j >> 2
        c0 = (j & (n_cc - 1)) * (GROUP * LANES)
        sls = [pl.ds(pl.multiple_of(c0 + k * LANES, LANES), LANES)
               for k in range(GROUP)]
        ps = [pb[pr, sl] for sl in sls]
        for b in range(B):
            xs = [xb[b, pr, sl] for sl in sls]
            for k in range(GROUP):
                xb[b, pr, sls[k]] = xs[k] + ps[k]
        return 0

    lax.fori_loop(0, CHUNK * n_cc, body, 0)


def _sc_add(x_hbm, p_hbm, o_hbm, *scr):
    xbufs = scr[:NBUF]
    pbufs = scr[NBUF:2 * NBUF]
    sin = scr[2 * NBUF:3 * NBUF]
    sout = scr[3 * NBUF:]
    wid = lax.axis_index("s") * 2 + lax.axis_index("c")
    s0 = wid * ROWS_W

    def start_in(c):
        buf = c % NBUF
        srow = s0 + c * CHUNK
        return [
            pltpu.async_copy(p_hbm.at[pl.ds(srow, CHUNK)],
                             pbufs[buf], sin[buf]),
            pltpu.async_copy(x_hbm.at[:, pl.ds(srow, CHUNK), :],
                             xbufs[buf], sin[buf]),
        ]

    def start_out(c):
        buf = c % NBUF
        srow = s0 + c * CHUNK
        return [
            pltpu.async_copy(xbufs[buf],
                             o_hbm.at[:, pl.ds(srow, CHUNK), :], sout[buf]),
        ]

    pend_in = {0: start_in(0), 1: start_in(1)}
    pend_out = {}
    for c in range(NCHUNK):
        buf = c % NBUF
        for cp in pend_in.pop(c):
            cp.wait()
        if c + 2 < NCHUNK:
            if c >= 1:
                for cp in pend_out.pop(c - 1):
                    cp.wait()
            pend_in[c + 2] = start_in(c + 2)
        _compute(xbufs[buf], pbufs[buf])
        pend_out[c] = start_out(c)
    for c, cps in sorted(pend_out.items()):
        for cp in cps:
            cp.wait()


@jax.jit
def _sc_kernel(x, p2d):
    mesh = plsc.VectorSubcoreMesh(core_axis_name="c", subcore_axis_name="s")
    run = pl.kernel(
        _sc_add,
        out_type=jax.ShapeDtypeStruct((B, S, D), jnp.float32),
        mesh=mesh,
        scratch_types=(
            [pltpu.VMEM((B, CHUNK, D), jnp.float32)] * NBUF
            + [pltpu.VMEM((CHUNK, D), jnp.float32)] * NBUF
            + [pltpu.SemaphoreType.DMA] * (2 * NBUF)
        ),
    )
    return run(x, p2d)


def kernel(x, pos_table):
    return _sc_kernel(x, pos_table)


# speedup vs baseline: 1.0824x; 1.0007x over previous
import jax
import jax.numpy as jnp
from jax import lax
from jax.experimental import pallas as pl
from jax.experimental.pallas import tpu as pltpu
from jax.experimental.pallas import tpu_sc as plsc

B = 4
S = 4096
D = 1024
NW = 32
ROWS_W = S // NW
CHUNK = 8
NCHUNK = ROWS_W // CHUNK
NBUF = 3
LANES = 16
GROUP = 16


def _compute(xb, pb):
    n_cc = D // (GROUP * LANES)

    def body(j, _):
        pr = j >> 2
        c0 = (j & (n_cc - 1)) * (GROUP * LANES)
        sls = [pl.ds(pl.multiple_of(c0 + k * LANES, LANES), LANES)
               for k in range(GROUP)]
        ps = [pb[pr, sl] for sl in sls]
        for b in range(B):
            xs = [xb[b, pr, sl] for sl in sls]
            for k in range(GROUP):
                xb[b, pr, sls[k]] = xs[k] + ps[k]
        return 0

    lax.fori_loop(0, CHUNK * n_cc, body, 0)


def _sc_add(x_hbm, p_hbm, o_hbm, *scr):
    xbufs = scr[:NBUF]
    pbufs = scr[NBUF:2 * NBUF]
    sin = scr[2 * NBUF:3 * NBUF]
    sout = scr[3 * NBUF:]
    wid = lax.axis_index("s") * 2 + lax.axis_index("c")
    s0 = wid * ROWS_W

    def start_in(c):
        buf = c % NBUF
        srow = s0 + c * CHUNK
        return [
            pltpu.async_copy(p_hbm.at[pl.ds(srow, CHUNK)],
                             pbufs[buf], sin[buf]),
            pltpu.async_copy(x_hbm.at[:, pl.ds(srow, CHUNK), :],
                             xbufs[buf], sin[buf]),
        ]

    def start_out(c):
        buf = c % NBUF
        srow = s0 + c * CHUNK
        return [
            pltpu.async_copy(xbufs[buf],
                             o_hbm.at[:, pl.ds(srow, CHUNK), :], sout[buf]),
        ]

    pend_in = {0: start_in(0), 1: start_in(1)}
    pend_out = {}
    for c in range(NCHUNK):
        buf = c % NBUF
        for cp in pend_in.pop(c):
            cp.wait()
        if c + 2 < NCHUNK:
            if c >= 1:
                for cp in pend_out.pop(c - 1):
                    cp.wait()
            pend_in[c + 2] = start_in(c + 2)
        _compute(xbufs[buf], pbufs[buf])
        pend_out[c] = start_out(c)
    for c, cps in sorted(pend_out.items()):
        for cp in cps:
            cp.wait()


@jax.jit
def _sc_kernel(x, p2d):
    mesh = plsc.VectorSubcoreMesh(core_axis_name="c", subcore_axis_name="s")
    run = pl.kernel(
        _sc_add,
        out_type=jax.ShapeDtypeStruct((B, S, D), jnp.float32),
        mesh=mesh,
        scratch_types=(
            [pltpu.VMEM((B, CHUNK, D), jnp.float32)] * NBUF
            + [pltpu.VMEM((CHUNK, D), jnp.float32)] * NBUF
            + [pltpu.SemaphoreType.DMA] * (2 * NBUF)
        ),
        compiler_params=pltpu.CompilerParams(
            skip_device_barrier=True,
            disable_bounds_checks=True,
            disable_semaphore_checks=True,
        ),
    )
    return run(x, p2d)


def kernel(x, pos_table):
    return _sc_kernel(x, pos_table)


# speedup vs baseline: 1.0828x; 1.0004x over previous
import jax
import jax.numpy as jnp
from jax import lax
from jax.experimental import pallas as pl
from jax.experimental.pallas import tpu as pltpu
from jax.experimental.pallas import tpu_sc as plsc

B = 4
S = 4096
D = 1024
NW = 32
ROWS_W = S // NW
CHUNK = 8
NCHUNK = ROWS_W // CHUNK
NBUF = 3
LANES = 16
GROUP = 16


def _compute(xb, pb):
    n_cc = D // (GROUP * LANES)

    def body(j, _):
        pr = j >> 2
        c0 = (j & (n_cc - 1)) * (GROUP * LANES)
        sls = [pl.ds(pl.multiple_of(c0 + k * LANES, LANES), LANES)
               for k in range(GROUP)]
        ps = [pb[pr, sl] for sl in sls]
        for b in range(B):
            xs = [xb[b, pr, sl] for sl in sls]
            for k in range(GROUP):
                xb[b, pr, sls[k]] = xs[k] + ps[k]
        return 0

    lax.fori_loop(0, CHUNK * n_cc, body, 0)


def _sc_add(x_hbm, p_hbm, o_hbm, *scr):
    xbufs = scr[:NBUF]
    pbufs = scr[NBUF:2 * NBUF]
    sin = scr[2 * NBUF:3 * NBUF]
    sout = scr[3 * NBUF:]
    wid = lax.axis_index("s") * 2 + lax.axis_index("c")
    s0 = wid * ROWS_W

    def start_in(c):
        buf = c % NBUF
        srow = s0 + c * CHUNK
        return [
            pltpu.async_copy(p_hbm.at[pl.ds(srow, CHUNK)],
                             pbufs[buf], sin[buf]),
            pltpu.async_copy(x_hbm.at[:, pl.ds(srow, CHUNK), :],
                             xbufs[buf], sin[buf]),
        ]

    def start_out(c):
        buf = c % NBUF
        srow = s0 + c * CHUNK
        return [
            pltpu.async_copy(xbufs[buf],
                             o_hbm.at[:, pl.ds(srow, CHUNK), :], sout[buf]),
        ]

    pend_in = {0: start_in(0), 1: start_in(1)}
    pend_out = {}
    for c in range(NCHUNK):
        buf = c % NBUF
        for cp in pend_in.pop(c):
            cp.wait()
        if c + 2 < NCHUNK:
            if c >= 1:
                for cp in pend_out.pop(c - 1):
                    cp.wait()
            pend_in[c + 2] = start_in(c + 2)
        _compute(xbufs[buf], pbufs[buf])
        pend_out[c] = start_out(c)
    for c, cps in sorted(pend_out.items()):
        for cp in cps:
            cp.wait()


@jax.jit
def _sc_kernel(x, p2d):
    mesh = plsc.VectorSubcoreMesh(core_axis_name="c", subcore_axis_name="s")
    run = pl.kernel(
        _sc_add,
        out_type=jax.ShapeDtypeStruct((B, S, D), jnp.float32),
        mesh=mesh,
        scratch_types=(
            [pltpu.VMEM((B, CHUNK, D), jnp.float32)] * NBUF
            + [pltpu.VMEM((CHUNK, D), jnp.float32)] * NBUF
            + [pltpu.SemaphoreType.DMA] * (2 * NBUF)
        ),
    )
    return run(x, p2d)


def kernel(x, pos_table):
    return _sc_kernel(x, pos_table)
